# baseline (speedup 1.0000x reference)
import functools

import jax
import jax.numpy as jnp
from jax import lax
from jax.experimental import pallas as pl
from jax.experimental.pallas import tpu as pltpu
from jax.experimental.pallas import tpu_sc as plsc

DIM_ = 1024
NE_ = 8
HID_ = 512
NTOK_ = 4096
NASN_ = 2 * NTOK_
EPAD_ = 128
RTB_ = 1024
BT_ = 256
P_ = NASN_ + NE_ * BT_
NB_ = P_ // BT_
NBPAD_ = 64


def _router_body(x_ref, wr_ref, br_ref, ew_ref, aux_ref, psum_ref):
    tb = pl.program_id(0)
    xf = x_ref[...]
    lg = jnp.dot(xf, wr_ref[...], precision=lax.Precision.DEFAULT,
                 preferred_element_type=jnp.float32)
    lg = (lg + br_ref[...]) * 10.0
    col = lax.broadcasted_iota(jnp.int32, (RTB_, EPAD_), 1)
    valid = col < NE_
    lg = jnp.where(valid, lg, -1e30)
    m1 = jnp.max(lg, axis=1, keepdims=True)
    a1 = jnp.min(jnp.where(lg == m1, col, EPAD_), axis=1, keepdims=True)
    lg2 = jnp.where(col == a1, -1e30, lg)
    m2 = jnp.max(lg2, axis=1, keepdims=True)
    a2 = jnp.min(jnp.where(lg2 == m2, col, EPAD_), axis=1, keepdims=True)
    w1w = 1.0 / (1.0 + jnp.exp(m2 - m1))
    ew_ref[...] = (jnp.where(col == 0, a1.astype(jnp.float32), 0.0)
                   + jnp.where(col == 1, a2.astype(jnp.float32), 0.0)
                   + jnp.where(col == 2, w1w, 0.0)
                   + jnp.where(col == 3, 1.0 - w1w, 0.0))
    ex = jnp.where(valid, jnp.exp(lg - m1), 0.0)
    p = ex / jnp.sum(ex, axis=1, keepdims=True)
    ps = jnp.sum(p, axis=0, keepdims=True)

    @pl.when(tb == 0)
    def _init():
        psum_ref[...] = jnp.zeros_like(psum_ref)

    psum_ref[...] += ps

    @pl.when(tb == pl.num_programs(0) - 1)
    def _fin():
        s = psum_ref[...]
        aux_ref[...] = (jnp.sum(s * s) / NE_ * 1e-5) * jnp.ones(
            (1, 1), jnp.float32)


def _count_sc_body(e_hbm, cnt_hbm, ev_v, cntv):
    w = lax.axis_index("s")
    nloc = NASN_ // 16
    abase = w * nloc
    pltpu.sync_copy(e_hbm.at[pl.ds(abase, nloc)], ev_v)
    iota = lax.iota(jnp.int32, 16)
    one16 = jnp.ones((16,), jnp.int32)
    zero16 = jnp.zeros((16,), jnp.int32)
    efull = [jnp.full((16,), e, jnp.int32) for e in range(NE_)]
    acc = [jnp.zeros((16,), jnp.int32) for _ in range(NE_)]
    for g in range(nloc // 16):
        ev = ev_v[pl.ds(g * 16, 16)]
        for e in range(NE_):
            acc[e] = acc[e] + jnp.where(ev == efull[e], one16, zero16)
    cnt16 = jnp.zeros((16,), jnp.int32)
    for e in range(NE_):
        tot = acc[e]
        for d in (1, 2, 4, 8):
            tot = tot + jnp.take(tot, iota ^ d)
        cnt16 = cnt16 + jnp.where(iota == efull[e], tot, zero16)
    cntv[...] = cnt16
    pltpu.sync_copy(cntv, cnt_hbm.at[w])


def _route_sc_body(e_hbm, cnt_hbm, pos_hbm, src_hbm, beid_hbm,
                   ev_v, posv, posv2, tokv2, cntv, allc, beidv):
    w = lax.axis_index("s")
    nloc = NASN_ // 16
    abase = w * nloc
    pltpu.sync_copy(e_hbm.at[pl.ds(abase, nloc)], ev_v)
    pltpu.sync_copy(cnt_hbm, allc)
    iota = lax.iota(jnp.int32, 16)
    one16 = jnp.ones((16,), jnp.int32)
    zero16 = jnp.zeros((16,), jnp.int32)
    efull = [jnp.full((16,), e, jnp.int32) for e in range(NE_)]

    total16 = jnp.zeros((16,), jnp.int32)
    pref16 = jnp.zeros((16,), jnp.int32)
    for w2 in range(16):
        row = allc[w2]
        total16 = total16 + row
        ind = jnp.full((16,), jnp.where(w2 < w, 1, 0), jnp.int32)
        pref16 = pref16 + row * ind
    padded16 = ((total16 + (BT_ - 1)) >> 8) << 8
    offs16 = jnp.zeros((16,), jnp.int32)
    for e in range(NE_):
        pe = jnp.take(padded16, efull[e])
        gt = jnp.where(iota > efull[e], one16, zero16)
        offs16 = offs16 + pe * gt
    start16 = offs16 + pref16

    @pl.when(w == 0)
    def _beid():
        for v in range(NBPAD_ // 16):
            bstart = (iota + 16 * v) * BT_
            acc2 = jnp.zeros((16,), jnp.int32)
            for e in range(NE_):
                offe = jnp.take(offs16, efull[e])
                acc2 = acc2 + jnp.where(offe <= bstart, one16, zero16)
            beidv[pl.ds(v * 16, 16)] = acc2 - one16
        pltpu.sync_copy(beidv, beid_hbm)

    cntv[...] = start16

    def _p2(g, _):
        ev = ev_v[pl.ds(g * 16, 16)]
        run16 = cntv[...]
        base16 = jnp.take(run16, ev)
        pref = jnp.zeros((16,), jnp.int32)
        for d in range(1, 16):
            shifted = jnp.take(ev, jnp.maximum(iota - d, 0))
            m1 = jnp.where(shifted == ev, one16, zero16)
            m2 = jnp.where(iota >= d, one16, zero16)
            pref = pref + m1 * m2
        upd = jnp.zeros((16,), jnp.int32)
        for e in range(NE_):
            ce = jnp.where(ev == efull[e], one16, zero16)
            for d in (1, 2, 4, 8):
                ce = ce + jnp.take(ce, iota ^ d)
            upd = upd + jnp.where(iota == efull[e], ce, zero16)
        posv[pl.ds(g * 16, 16)] = base16 + pref
        cntv[...] = run16 + upd
        return 0

    lax.fori_loop(0, nloc // 16, _p2, 0)

    pltpu.sync_copy(posv, pos_hbm.at[pl.ds(abase, nloc)])

    tbase = abase % NTOK_
    for j in range(4):
        for c in range(8):
            sl = pl.ds(c * 16, 16)
            posv2[j, sl] = posv[pl.ds(j * 128 + c * 16, 16)]
            tokv2[j, sl] = jnp.full((16,), tbase + j * 128 + c * 16,
                                    jnp.int32) + iota
    for j in range(4):
        pltpu.sync_copy(tokv2.at[j], src_hbm.at[posv2.at[j]])


def _gather_sc_body(x3_hbm, src_hbm, xs_hbm, idxv, rows, sem):
    c = lax.axis_index("c")
    s = lax.axis_index("s")
    wid = s * 2 + c
    base = wid * (P_ // 32)
    for t in range(P_ // 32 // 64):
        off = base + t * 64
        pltpu.sync_copy(src_hbm.at[pl.ds(off, 64)], idxv)
        for q in range(4):
            sl = pl.ds(q * 16, 16)
            v = idxv[sl]
            idxv[sl] = jnp.minimum(jnp.maximum(v, 0), NTOK_ - 1)
        pltpu.async_copy(x3_hbm.at[idxv], rows, sem).wait()
        pltpu.sync_copy(rows, xs_hbm.at[pl.ds(off, 64)])


def _group_body(beid_ref, xs_ref, w1_ref, b1_ref, w2_ref, b2_ref, out_ref):
    h = jnp.dot(xs_ref[...], w1_ref[0],
                preferred_element_type=jnp.float32) + b1_ref[0]
    h = h * (1.0 / (1.0 + jnp.exp(-h)))
    out_ref[...] = jnp.dot(h.astype(jnp.bfloat16), w2_ref[0],
                           preferred_element_type=jnp.float32) + b2_ref[0]


def _combine_sc_body(outs3_hbm, pos_hbm, wtop_hbm, out3_hbm,
                     i0v, i1v, w0v, w1v, r0, r1, ov, sem):
    c = lax.axis_index("c")
    s = lax.axis_index("s")
    wid = s * 2 + c
    t0 = wid * (NTOK_ // 32)
    for ch in range(NTOK_ // 32 // 16):
        tb = t0 + ch * 16
        pltpu.sync_copy(pos_hbm.at[pl.ds(tb, 16)], i0v)
        pltpu.sync_copy(pos_hbm.at[pl.ds(NTOK_ + tb, 16)], i1v)
        pltpu.sync_copy(wtop_hbm.at[0, pl.ds(tb, 16)], w0v)
        pltpu.sync_copy(wtop_hbm.at[1, pl.ds(tb, 16)], w1v)
        pltpu.async_copy(outs3_hbm.at[i0v], r0, sem).wait()
        pltpu.async_copy(outs3_hbm.at[i1v], r1, sem).wait()

        w0vec = w0v[...]
        w1vec = w1v[...]

        def _tok(j, _):
            jf = jnp.full((16,), j, jnp.int32)
            w0b = jnp.take(w0vec, jf)
            w1b = jnp.take(w1vec, jf)
            for sub in range(8):
                for li in range(8):
                    sl = pl.ds(li * 16, 16)
                    ov[j, sub, sl] = (r0[j, sub, sl] * w0b
                                      + r1[j, sub, sl] * w1b)
            return 0

        lax.fori_loop(0, 16, _tok, 0)
        pltpu.sync_copy(ov, out3_hbm.at[pl.ds(tb, 16)])


_count_sc = functools.partial(
    pl.kernel,
    out_type=jax.ShapeDtypeStruct((16, 16), jnp.int32),
    mesh=plsc.VectorSubcoreMesh(core_axis_name="c", subcore_axis_name="s",
                                num_cores=1),
    scratch_types=[
        pltpu.VMEM((NASN_ // 16,), jnp.int32),
        pltpu.VMEM((16,), jnp.int32),
    ],
)(_count_sc_body)

_route_sc = functools.partial(
    pl.kernel,
    out_type=[
        jax.ShapeDtypeStruct((NASN_,), jnp.int32),
        jax.ShapeDtypeStruct((P_,), jnp.int32),
        jax.ShapeDtypeStruct((NBPAD_,), jnp.int32),
    ],
    mesh=plsc.VectorSubcoreMesh(core_axis_name="c", subcore_axis_name="s",
                                num_cores=1),
    scratch_types=[
        pltpu.VMEM((NASN_ // 16,), jnp.int32),
        pltpu.VMEM((NASN_ // 16,), jnp.int32),
        pltpu.VMEM((4, 128), jnp.int32),
        pltpu.VMEM((4, 128), jnp.int32),
        pltpu.VMEM((16,), jnp.int32),
        pltpu.VMEM((16, 16), jnp.int32),
        pltpu.VMEM((NBPAD_,), jnp.int32),
    ],
)(_route_sc_body)

_gather_sc = functools.partial(
    pl.kernel,
    out_type=jax.ShapeDtypeStruct((P_, 4, 128), jnp.int32),
    mesh=plsc.VectorSubcoreMesh(core_axis_name="c", subcore_axis_name="s"),
    scratch_types=[
        pltpu.VMEM((64,), jnp.int32),
        pltpu.VMEM((64, 4, 128), jnp.int32),
        pltpu.SemaphoreType.DMA,
    ],
)(_gather_sc_body)

_combine_sc = functools.partial(
    pl.kernel,
    out_type=jax.ShapeDtypeStruct((NTOK_, 8, 128), jnp.float32),
    mesh=plsc.VectorSubcoreMesh(core_axis_name="c", subcore_axis_name="s"),
    scratch_types=[
        pltpu.VMEM((16,), jnp.int32),
        pltpu.VMEM((16,), jnp.int32),
        pltpu.VMEM((16,), jnp.float32),
        pltpu.VMEM((16,), jnp.float32),
        pltpu.VMEM((16, 8, 128), jnp.float32),
        pltpu.VMEM((16, 8, 128), jnp.float32),
        pltpu.VMEM((16, 8, 128), jnp.float32),
        pltpu.SemaphoreType.DMA,
    ],
)(_combine_sc_body)


@jax.jit
def kernel(x, Wr, br, W1, b1, W2, b2):
    B, S, D = x.shape
    x_flat = x.reshape(-1, D)
    wr_pad = jnp.zeros((D, EPAD_), jnp.float32).at[:, :NE_].set(Wr)
    br_pad = jnp.zeros((1, EPAD_), jnp.float32).at[0, :NE_].set(br)

    ew, aux = pl.pallas_call(
        _router_body,
        grid=(NTOK_ // RTB_,),
        in_specs=[
            pl.BlockSpec((RTB_, DIM_), lambda t: (t, 0)),
            pl.BlockSpec((DIM_, EPAD_), lambda t: (0, 0)),
            pl.BlockSpec((1, EPAD_), lambda t: (0, 0)),
        ],
        out_specs=[
            pl.BlockSpec((RTB_, EPAD_), lambda t: (t, 0)),
            pl.BlockSpec((1, 1), lambda t: (0, 0)),
        ],
        out_shape=[
            jax.ShapeDtypeStruct((NTOK_, EPAD_), jnp.float32),
            jax.ShapeDtypeStruct((1, 1), jnp.float32),
        ],
        scratch_shapes=[pltpu.VMEM((1, EPAD_), jnp.float32)],
        compiler_params=pltpu.CompilerParams(
            dimension_semantics=("arbitrary",)),
    )(x_flat, wr_pad, br_pad)

    cols4 = ew[:, :4].transpose()
    e_all = jnp.concatenate([cols4[0], cols4[1]]).astype(jnp.int32)
    wtop = jnp.stack([cols4[2], cols4[3]])

    cnts = _count_sc(e_all)
    pos, src, beid = _route_sc(e_all, cnts)
    x_i32 = lax.bitcast_convert_type(
        x_flat.astype(jnp.bfloat16).reshape(NTOK_, 512, 2),
        jnp.int32).reshape(NTOK_, 4, 128)
    xs_i32 = _gather_sc(x_i32, src)
    xs3 = lax.bitcast_convert_type(
        xs_i32.reshape(P_, 512), jnp.bfloat16).reshape(P_, DIM_)

    grid_spec = pltpu.PrefetchScalarGridSpec(
        num_scalar_prefetch=1,
        grid=(NB_,),
        in_specs=[
            pl.BlockSpec((BT_, DIM_), lambda b, beid: (b, 0)),
            pl.BlockSpec((1, DIM_, HID_), lambda b, beid: (beid[b], 0, 0)),
            pl.BlockSpec((1, 1, HID_), lambda b, beid: (beid[b], 0, 0)),
            pl.BlockSpec((1, HID_, DIM_), lambda b, beid: (beid[b], 0, 0)),
            pl.BlockSpec((1, 1, DIM_), lambda b, beid: (beid[b], 0, 0)),
        ],
        out_specs=pl.BlockSpec((BT_, DIM_), lambda b, beid: (b, 0)),
    )
    outs = pl.pallas_call(
        _group_body,
        grid_spec=grid_spec,
        out_shape=jax.ShapeDtypeStruct((P_, DIM_), jnp.float32),
        compiler_params=pltpu.CompilerParams(
            dimension_semantics=("arbitrary",)),
    )(beid, xs3, W1.astype(jnp.bfloat16),
      b1.reshape(NE_, 1, HID_), W2.astype(jnp.bfloat16),
      b2.reshape(NE_, 1, DIM_))

    out3 = _combine_sc(outs.reshape(P_, 8, 128), pos, wtop)
    return out3.reshape(B, S, D), aux.reshape(())
